# baseline (speedup 1.0000x reference)
import functools

import jax

jax.config.update('jax_enable_x64', True)

import jax.numpy as jnp
import numpy as np
from jax import lax
from jax.experimental import pallas as pl
from jax.experimental.pallas import tpu as pltpu
from jax.experimental.pallas import tpu_sc as plsc

_B, _S = 4, 8192
_MAX_MASKED = 1229
_NVEC = _S // 16


_U32, _U64 = np.uint32, np.uint64


def _threefry2x32(k0, k1, x0, x1):
    x0 = x0.copy()
    x1 = x1.copy()
    ks = [_U32(k0), _U32(k1), _U32(k0) ^ _U32(k1) ^ _U32(0x1BD11BDA)]
    rotations = [(13, 15, 26, 6), (17, 29, 16, 24)]
    with np.errstate(over='ignore'):
        x0 = x0 + ks[0]
        x1 = x1 + ks[1]
        for i in range(5):
            for r in rotations[i % 2]:
                x0 = x0 + x1
                x1 = (x1 << _U32(r)) | (x1 >> _U32(32 - r))
                x1 = x0 ^ x1
            x0 = x0 + ks[(i + 1) % 3]
            x1 = x1 + ks[(i + 2) % 3] + _U32(i + 1)
    return x0, x1


def _iota_2x32(n):
    c = np.arange(n, dtype=_U64)
    return (c >> _U64(32)).astype(_U32), (c & _U64(0xFFFFFFFF)).astype(_U32)


def _split_key(keypair, num):
    c1, c2 = _iota_2x32(num)
    b1, b2 = _threefry2x32(keypair[0], keypair[1], c1, c2)
    return np.stack([b1, b2], axis=1)


def _bits64(keypair, size):
    c1, c2 = _iota_2x32(size)
    b1, b2 = _threefry2x32(keypair[0], keypair[1], c1, c2)
    return (b1.astype(_U64) << _U64(32)) | b2.astype(_U64)


def _uniform_f64(keypair, shape):
    bits = _bits64(keypair, int(np.prod(shape)))
    fb = (bits >> _U64(12)) | np.float64(1.0).view(_U64)
    return np.maximum(0.0, fb.view(np.float64) - 1.0).reshape(shape)


def _randint64(keypair, shape, minval, maxval):
    size = int(np.prod(shape))
    k1, k2 = _split_key(keypair, 2)
    higher, lower = _bits64(k1, size), _bits64(k2, size)
    span = _U64(maxval - minval)
    mult = _U64(pow(2, 32, int(span)))
    mult = (mult * mult) % span
    offset = ((higher % span) * mult + (lower % span)) % span
    return (offset.astype(np.int64) + minval).reshape(shape)


def _derive_constants():
    root = np.array([0, 42], _U32)
    k1, k2, k3, k4 = _split_key(root, 4)
    rand = _uniform_f64(k1, (_B, _S))
    rtp = _uniform_f64(k2, (_B, _S)) < 0.1
    randtok = _randint64(k3, (_B, _S), 0, 32000)
    repl = _uniform_f64(k4, (_B, _S)) < 0.9
    perm = np.argsort(-rand, axis=-1, kind='stable').astype(np.int32)
    rtp &= ~np.isin(randtok, (0, 1, 2))
    ov = np.where(rtp, randtok, np.where(repl, -2, -1)).astype(np.int32)
    ovp = np.take_along_axis(ov, perm, axis=-1)
    pk = perm | ((ovp + 2) << 13)
    return pk, ov


_PK, _OV = _derive_constants()


def _mlm_sc(seq_hbm, pk_hbm, ov_hbm, out_hbm, lab_hbm,
            seqv, pkv, ovv, csv, offsv, icsv, ioffsv, outv, labv,
            sem1, sem2):
    wid = lax.axis_index("s")
    i32 = jnp.int32

    @pl.when(wid < _B)
    def _row():
        row = wid
        cp1 = pltpu.async_copy(seq_hbm.at[row], seqv, sem1)
        cp2 = pltpu.async_copy(pk_hbm.at[row], pkv, sem2)
        cp1.wait()
        cp2.wait()

        lane = lax.iota(jnp.int32, 16)

        def _splat(ref, v):
            return plsc.load_gather(ref, [jnp.full((16,), v, jnp.int32)])

        @plsc.parallel_loop(np.int32(0), np.int32(_NVEC), step=np.int32(1), unroll=8)
        def _pass_a(v):
            v = lax.convert_element_type(v, jnp.int32)
            idx = pkv[pl.ds(v * np.int32(16), 16)] & i32(0x1FFF)
            s = plsc.load_gather(seqv, [idx])
            c = jnp.where(s > i32(2), i32(1), i32(0))
            csv[pl.ds(v * np.int32(16), 16)] = jnp.cumsum(c, dtype=jnp.int32)

        @plsc.parallel_loop(np.int32(0), np.int32(80), step=np.int32(1), unroll=4)
        def _pass_a2(v):
            v = lax.convert_element_type(v, jnp.int32)
            s = seqv[pl.ds(v * np.int32(16), 16)]
            m = jnp.where(s > i32(2), i32(1), i32(0))
            icsv[pl.ds(v * np.int32(16), 16)] = jnp.cumsum(m, dtype=jnp.int32)

        def p_b(g, carry):
            tidx = g * np.int32(256) + lane * np.int32(16) + np.int32(15)
            tot = plsc.load_gather(csv, [tidx])
            cst = jnp.cumsum(tot, dtype=jnp.int32) + carry
            offsv[pl.ds(g * np.int32(16), 16)] = cst - tot
            return carry + jnp.sum(tot, dtype=jnp.int32)

        C = lax.fori_loop(i32(0), i32(_NVEC // 16), p_b, i32(0))

        def p_b2(g, carry):
            tidx = g * np.int32(256) + lane * np.int32(16) + np.int32(15)
            tot = plsc.load_gather(icsv, [tidx])
            cst = jnp.cumsum(tot, dtype=jnp.int32) + carry
            ioffsv[pl.ds(g * np.int32(16), 16)] = cst - tot
            return carry + jnp.sum(tot, dtype=jnp.int32)

        lax.fori_loop(i32(0), i32(5), p_b2, i32(0))

        p = C.astype(jnp.float32) * np.float32(0.15)
        Ti = p.astype(jnp.int32)
        Ti = Ti + jnp.where(Ti.astype(jnp.float32) < p, i32(1), i32(0))

        @plsc.parallel_loop(np.int32(0), np.int32(77), step=np.int32(1), unroll=4, carry=jnp.zeros((16,), jnp.int32))
        def kacc(v, acc):
            v = lax.convert_element_type(v, jnp.int32)
            csg = icsv[pl.ds(v * np.int32(16), 16)] + _splat(ioffsv, v)
            valid = (v * np.int32(16) + lane) < i32(_MAX_MASKED)
            return acc + jnp.where(valid & (csg <= Ti), i32(1), i32(0))

        K = jnp.sum(kacc, dtype=jnp.int32)

        @plsc.parallel_loop(np.int32(0), np.int32(_NVEC), step=np.int32(1), unroll=8)
        def _pass_c(v):
            v = lax.convert_element_type(v, jnp.int32)
            w = pkv[pl.ds(v * np.int32(16), 16)]
            idx = w & i32(0x1FFF)
            o = (w >> i32(13)) - i32(2)
            s = plsc.load_gather(seqv, [idx])
            csg = csv[pl.ds(v * np.int32(16), 16)] + _splat(offsv, v)
            selb = (s > i32(2)) & (csg <= K)
            outp = jnp.where(o >= i32(0), o,
                             jnp.where((o == i32(-2)) & selb, i32(2), s))
            labp = jnp.where(selb, s, i32(0))
            plsc.store_scatter(outv, [idx], outp)
            plsc.store_scatter(labv, [idx], labp)

        @pl.when(K > C)
        def _overflow():
            pltpu.sync_copy(ov_hbm.at[row], ovv)

            def p4(v, run):
                s = seqv[pl.ds(v * np.int32(16), 16)]
                nc = jnp.where(s > i32(2), i32(0), i32(1))
                ncs = jnp.cumsum(nc, dtype=jnp.int32) + run
                extra = (nc > i32(0)) & (C + ncs <= K)
                o = ovv[pl.ds(v * np.int32(16), 16)]
                outv[pl.ds(v * np.int32(16), 16)] = jnp.where(
                    extra & (o == i32(-2)), i32(2),
                    outv[pl.ds(v * np.int32(16), 16)])
                labv[pl.ds(v * np.int32(16), 16)] = jnp.where(
                    extra, s, labv[pl.ds(v * np.int32(16), 16)])
                return run + jnp.sum(nc, dtype=jnp.int32)

            lax.fori_loop(i32(0), i32(_NVEC), p4, i32(0))

        pltpu.sync_copy(outv, out_hbm.at[row])
        pltpu.sync_copy(labv, lab_hbm.at[row])


@functools.cache
def _build_mlm_sc():
    return pl.kernel(
        _mlm_sc,
        out_type=(
            jax.ShapeDtypeStruct((_B, _S), jnp.int32),
            jax.ShapeDtypeStruct((_B, _S), jnp.int32),
        ),
        mesh=plsc.VectorSubcoreMesh(core_axis_name="c", subcore_axis_name="s", num_cores=1),
        compiler_params=pltpu.CompilerParams(needs_layout_passes=False),
        scratch_types=[
            pltpu.VMEM((_S,), jnp.int32),
            pltpu.VMEM((_S,), jnp.int32),
            pltpu.VMEM((_S,), jnp.int32),
            pltpu.VMEM((_S,), jnp.int32),
            pltpu.VMEM((_NVEC,), jnp.int32),
            pltpu.VMEM((1280,), jnp.int32),
            pltpu.VMEM((80,), jnp.int32),
            pltpu.VMEM((_S,), jnp.int32),
            pltpu.VMEM((_S,), jnp.int32),
            pltpu.SemaphoreType.DMA,
            pltpu.SemaphoreType.DMA,
        ],
    )


def kernel(seq):
    seq32 = seq.astype(jnp.int32)
    out32, lab32 = _build_mlm_sc()(seq32, jnp.asarray(_PK), jnp.asarray(_OV))
    return out32.astype(jnp.int64), lab32.astype(jnp.int64)


# speedup vs baseline: 1.0288x; 1.0288x over previous
import functools

import jax

jax.config.update('jax_enable_x64', True)

import jax.numpy as jnp
import numpy as np
from jax import lax
from jax.experimental import pallas as pl
from jax.experimental.pallas import tpu as pltpu
from jax.experimental.pallas import tpu_sc as plsc

_B, _S = 4, 8192
_MAX_MASKED = 1229
_NVEC = _S // 16


_U32, _U64 = np.uint32, np.uint64


def _threefry2x32(k0, k1, x0, x1):
    x0 = x0.copy()
    x1 = x1.copy()
    ks = [_U32(k0), _U32(k1), _U32(k0) ^ _U32(k1) ^ _U32(0x1BD11BDA)]
    rotations = [(13, 15, 26, 6), (17, 29, 16, 24)]
    with np.errstate(over='ignore'):
        x0 = x0 + ks[0]
        x1 = x1 + ks[1]
        for i in range(5):
            for r in rotations[i % 2]:
                x0 = x0 + x1
                x1 = (x1 << _U32(r)) | (x1 >> _U32(32 - r))
                x1 = x0 ^ x1
            x0 = x0 + ks[(i + 1) % 3]
            x1 = x1 + ks[(i + 2) % 3] + _U32(i + 1)
    return x0, x1


def _iota_2x32(n):
    c = np.arange(n, dtype=_U64)
    return (c >> _U64(32)).astype(_U32), (c & _U64(0xFFFFFFFF)).astype(_U32)


def _split_key(keypair, num):
    c1, c2 = _iota_2x32(num)
    b1, b2 = _threefry2x32(keypair[0], keypair[1], c1, c2)
    return np.stack([b1, b2], axis=1)


def _bits64(keypair, size):
    c1, c2 = _iota_2x32(size)
    b1, b2 = _threefry2x32(keypair[0], keypair[1], c1, c2)
    return (b1.astype(_U64) << _U64(32)) | b2.astype(_U64)


def _uniform_f64(keypair, shape):
    bits = _bits64(keypair, int(np.prod(shape)))
    fb = (bits >> _U64(12)) | np.float64(1.0).view(_U64)
    return np.maximum(0.0, fb.view(np.float64) - 1.0).reshape(shape)


def _randint64(keypair, shape, minval, maxval):
    size = int(np.prod(shape))
    k1, k2 = _split_key(keypair, 2)
    higher, lower = _bits64(k1, size), _bits64(k2, size)
    span = _U64(maxval - minval)
    mult = _U64(pow(2, 32, int(span)))
    mult = (mult * mult) % span
    offset = ((higher % span) * mult + (lower % span)) % span
    return (offset.astype(np.int64) + minval).reshape(shape)


def _derive_constants():
    root = np.array([0, 42], _U32)
    k1, k2, k3, k4 = _split_key(root, 4)
    rand = _uniform_f64(k1, (_B, _S))
    rtp = _uniform_f64(k2, (_B, _S)) < 0.1
    randtok = _randint64(k3, (_B, _S), 0, 32000)
    repl = _uniform_f64(k4, (_B, _S)) < 0.9
    perm = np.argsort(-rand, axis=-1, kind='stable').astype(np.int32)
    rtp &= ~np.isin(randtok, (0, 1, 2))
    ov = np.where(rtp, randtok, np.where(repl, -2, -1)).astype(np.int32)
    ovp = np.take_along_axis(ov, perm, axis=-1)
    pk = perm | ((ovp + 2) << 13)
    return pk, ov


_PK, _OV = _derive_constants()


def _mlm_sc(seq_hbm, pk_hbm, ov_hbm, out_hbm, lab_hbm,
            seqv, pkv, ovv, csv, offsv, icsv, ioffsv, outv, labv,
            sem1, sem2):
    wid = lax.axis_index("s")
    i32 = jnp.int32

    @pl.when(wid < _B)
    def _row():
        row = wid
        cp1 = pltpu.async_copy(seq_hbm.at[row], seqv, sem1)
        cp2 = pltpu.async_copy(pk_hbm.at[row], pkv, sem2)
        cp1.wait()
        cp2.wait()

        lane = lax.iota(jnp.int32, 16)

        def _splat(ref, v):
            return plsc.load_gather(ref, [jnp.full((16,), v, jnp.int32)])

        @plsc.parallel_loop(np.int32(0), np.int32(_NVEC), step=np.int32(1), unroll=4)
        def _pass_a(v):
            v = lax.convert_element_type(v, jnp.int32)
            idx = pkv[pl.ds(v * np.int32(16), 16)] & i32(0x1FFF)
            s = plsc.load_gather(seqv, [idx])
            c = jnp.where(s > i32(2), i32(1), i32(0))
            csv[pl.ds(v * np.int32(16), 16)] = jnp.cumsum(c, dtype=jnp.int32)

        @plsc.parallel_loop(np.int32(0), np.int32(80), step=np.int32(1), unroll=4)
        def _pass_a2(v):
            v = lax.convert_element_type(v, jnp.int32)
            s = seqv[pl.ds(v * np.int32(16), 16)]
            m = jnp.where(s > i32(2), i32(1), i32(0))
            icsv[pl.ds(v * np.int32(16), 16)] = jnp.cumsum(m, dtype=jnp.int32)

        def p_b(g, carry):
            tidx = g * np.int32(256) + lane * np.int32(16) + np.int32(15)
            tot = plsc.load_gather(csv, [tidx])
            cst = jnp.cumsum(tot, dtype=jnp.int32) + carry
            offsv[pl.ds(g * np.int32(16), 16)] = cst - tot
            return carry + jnp.sum(tot, dtype=jnp.int32)

        C = lax.fori_loop(i32(0), i32(_NVEC // 16), p_b, i32(0))

        def p_b2(g, carry):
            tidx = g * np.int32(256) + lane * np.int32(16) + np.int32(15)
            tot = plsc.load_gather(icsv, [tidx])
            cst = jnp.cumsum(tot, dtype=jnp.int32) + carry
            ioffsv[pl.ds(g * np.int32(16), 16)] = cst - tot
            return carry + jnp.sum(tot, dtype=jnp.int32)

        lax.fori_loop(i32(0), i32(5), p_b2, i32(0))

        p = C.astype(jnp.float32) * np.float32(0.15)
        Ti = p.astype(jnp.int32)
        Ti = Ti + jnp.where(Ti.astype(jnp.float32) < p, i32(1), i32(0))

        @plsc.parallel_loop(np.int32(0), np.int32(77), step=np.int32(1), unroll=4, carry=jnp.zeros((16,), jnp.int32))
        def kacc(v, acc):
            v = lax.convert_element_type(v, jnp.int32)
            csg = icsv[pl.ds(v * np.int32(16), 16)] + _splat(ioffsv, v)
            valid = (v * np.int32(16) + lane) < i32(_MAX_MASKED)
            return acc + jnp.where(valid & (csg <= Ti), i32(1), i32(0))

        K = jnp.sum(kacc, dtype=jnp.int32)

        @plsc.parallel_loop(np.int32(0), np.int32(_NVEC), step=np.int32(1), unroll=4)
        def _pass_c(v):
            v = lax.convert_element_type(v, jnp.int32)
            w = pkv[pl.ds(v * np.int32(16), 16)]
            idx = w & i32(0x1FFF)
            o = (w >> i32(13)) - i32(2)
            s = plsc.load_gather(seqv, [idx])
            csg = csv[pl.ds(v * np.int32(16), 16)] + _splat(offsv, v)
            selb = (s > i32(2)) & (csg <= K)
            outp = jnp.where(o >= i32(0), o,
                             jnp.where((o == i32(-2)) & selb, i32(2), s))
            labp = jnp.where(selb, s, i32(0))
            plsc.store_scatter(outv, [idx], outp)
            plsc.store_scatter(labv, [idx], labp)

        @pl.when(K > C)
        def _overflow():
            pltpu.sync_copy(ov_hbm.at[row], ovv)

            def p4(v, run):
                s = seqv[pl.ds(v * np.int32(16), 16)]
                nc = jnp.where(s > i32(2), i32(0), i32(1))
                ncs = jnp.cumsum(nc, dtype=jnp.int32) + run
                extra = (nc > i32(0)) & (C + ncs <= K)
                o = ovv[pl.ds(v * np.int32(16), 16)]
                outv[pl.ds(v * np.int32(16), 16)] = jnp.where(
                    extra & (o == i32(-2)), i32(2),
                    outv[pl.ds(v * np.int32(16), 16)])
                labv[pl.ds(v * np.int32(16), 16)] = jnp.where(
                    extra, s, labv[pl.ds(v * np.int32(16), 16)])
                return run + jnp.sum(nc, dtype=jnp.int32)

            lax.fori_loop(i32(0), i32(_NVEC), p4, i32(0))

        pltpu.sync_copy(outv, out_hbm.at[row])
        pltpu.sync_copy(labv, lab_hbm.at[row])


@functools.cache
def _build_mlm_sc():
    return pl.kernel(
        _mlm_sc,
        out_type=(
            jax.ShapeDtypeStruct((_B, _S), jnp.int32),
            jax.ShapeDtypeStruct((_B, _S), jnp.int32),
        ),
        mesh=plsc.VectorSubcoreMesh(core_axis_name="c", subcore_axis_name="s", num_cores=1),
        compiler_params=pltpu.CompilerParams(needs_layout_passes=False),
        scratch_types=[
            pltpu.VMEM((_S,), jnp.int32),
            pltpu.VMEM((_S,), jnp.int32),
            pltpu.VMEM((_S,), jnp.int32),
            pltpu.VMEM((_S,), jnp.int32),
            pltpu.VMEM((_NVEC,), jnp.int32),
            pltpu.VMEM((1280,), jnp.int32),
            pltpu.VMEM((80,), jnp.int32),
            pltpu.VMEM((_S,), jnp.int32),
            pltpu.VMEM((_S,), jnp.int32),
            pltpu.SemaphoreType.DMA,
            pltpu.SemaphoreType.DMA,
        ],
    )


def kernel(seq):
    seq32 = seq.astype(jnp.int32)
    out32, lab32 = _build_mlm_sc()(seq32, jnp.asarray(_PK), jnp.asarray(_OV))
    return out32.astype(jnp.int64), lab32.astype(jnp.int64)
